# baseline (speedup 1.0000x reference)
import dataclasses
import functools

import jax
import jax.numpy as jnp
from jax import lax
from jax.experimental import pallas as pl
from jax.experimental.pallas import tpu as pltpu
from jax.experimental.pallas import tpu_sc as plsc

EPS = 0.3
L = 16


def _sc_compiler_params():
    cp = pltpu.CompilerParams()
    fields = pltpu.CompilerParams.__dataclass_fields__
    if "needs_layout_passes" in fields:
        cp = dataclasses.replace(cp, needs_layout_passes=False)
    if "use_tc_tiling_on_sc" in fields:
        cp = dataclasses.replace(cp, use_tc_tiling_on_sc=False)
    return cp


def _rsqrt_newton(x):
    i = plsc.bitcast(x, jnp.int32)
    i = jnp.int32(0x5F3759DF) - (i >> 1)
    y = plsc.bitcast(i, jnp.float32)
    for _ in range(3):
        y = y * (1.5 - 0.5 * x * y * y)
    return y


def _sc_degree(dst2d, n_nodes):
    KK, C = dst2d.shape
    E = KK * C
    NT = 16
    RPT2 = KK // NT
    FB = 10
    NPAD = ((n_nodes + NT * C - 1) // (NT * C)) * (NT * C)
    RPT = NPAD // NT
    OPT = n_nodes // NT
    assert RPT2 * NT == KK and OPT * NT == n_nodes and RPT2 % FB == 0

    mesh = plsc.VectorSubcoreMesh(core_axis_name="c", subcore_axis_name="s")

    @functools.partial(
        pl.kernel, mesh=mesh,
        out_type=jax.ShapeDtypeStruct((n_nodes, L), jnp.float32),
        scratch_types=[
            pltpu.VMEM_SHARED((NPAD, L), jnp.float32),
            pltpu.VMEM((RPT2, C), jnp.int32),
            pltpu.VMEM((C, L), jnp.float32),
            pltpu.VMEM((OPT, L), jnp.float32),
            pltpu.SemaphoreType.DMA,
        ],
        compiler_params=_sc_compiler_params(),
    )
    def deg_kernel(dst_h, out_h, acc, idx_v, ones_v, buf_v, sem):
        cid = lax.axis_index("c")
        sid = lax.axis_index("s")

        @pl.when(cid == 0)
        def _():
            zv = jnp.zeros((L,), jnp.float32)

            @pl.loop(0, C)
            def _(j):
                ones_v[j, :] = zv

            @pl.loop(0, RPT // C)
            def _(r):
                pltpu.sync_copy(ones_v, acc.at[pl.ds(sid * RPT + r * C, C)])

            ov = jnp.ones((L,), jnp.float32)

            @pl.loop(0, C)
            def _(j):
                ones_v[j, :] = ov

            pltpu.sync_copy(dst_h.at[pl.ds(sid * RPT2, RPT2)], idx_v)
            plsc.subcore_barrier()

            @pl.loop(0, RPT2 // FB)
            def _(g):
                for j in range(FB):
                    pltpu.async_copy(ones_v, acc.at[idx_v.at[g * FB + j]],
                                     sem, add=True)
                for j in range(FB):
                    pltpu.make_async_copy(
                        ones_v, acc.at[idx_v.at[g * FB + j]], sem).wait()

            plsc.subcore_barrier()
            pltpu.sync_copy(acc.at[pl.ds(sid * OPT, OPT)], buf_v)

            @pl.loop(0, OPT)
            def _(j):
                x = buf_v[j, :]
                x = jnp.maximum(x, 1.0)
                buf_v[j, :] = _rsqrt_newton(x)

            pltpu.sync_copy(buf_v, out_h.at[pl.ds(sid * OPT, OPT)])

    return deg_kernel(dst2d)


def _sc_weights(a, b, src2d, dst2d):
    N = a.shape[0]
    KK, C = src2d.shape
    NW = 32
    NT = 16
    RPW = KK // NW
    assert RPW * NW == KK

    mesh = plsc.VectorSubcoreMesh(core_axis_name="c", subcore_axis_name="s")

    @functools.partial(
        pl.kernel, mesh=mesh,
        out_type=jax.ShapeDtypeStruct((KK, C), jnp.float32),
        scratch_types=[
            pltpu.VMEM((N,), jnp.float32),
            pltpu.VMEM((N,), jnp.float32),
            pltpu.VMEM((RPW, C), jnp.int32),
            pltpu.VMEM((RPW, C), jnp.int32),
            pltpu.VMEM((RPW, C), jnp.float32),
            pltpu.SemaphoreType.DMA,
        ],
        compiler_params=_sc_compiler_params(),
    )
    def weights_kernel(a_h, b_h, src_h, dst_h, w_h,
                       a_v, b_v, si_v, di_v, w_v, sem):
        cid = lax.axis_index("c")
        sid = lax.axis_index("s")
        wid = cid * NT + sid
        cps = [
            pltpu.make_async_copy(a_h, a_v, sem),
            pltpu.make_async_copy(b_h, b_v, sem),
            pltpu.make_async_copy(src_h.at[pl.ds(wid * RPW, RPW)], si_v, sem),
            pltpu.make_async_copy(dst_h.at[pl.ds(wid * RPW, RPW)], di_v, sem),
        ]
        for cp in cps:
            cp.start()
        for cp in cps:
            cp.wait()

        @plsc.parallel_loop(0, RPW, unroll=2)
        def _(r):
            for o in range(C // L):
                s16 = si_v[r, pl.ds(o * L, L)]
                t16 = di_v[r, pl.ds(o * L, L)]
                av = plsc.load_gather(a_v, [t16])
                bv = plsc.load_gather(b_v, [s16])
                w_v[r, pl.ds(o * L, L)] = (
                    1.0 - 2.0 / (1.0 + jnp.exp(2.0 * (av + bv))))

        pltpu.sync_copy(w_v, w_h.at[pl.ds(wid * RPW, RPW)])

    return weights_kernel(a, b, src2d, dst2d)


def _sc_layer(table, w2d, src2d, dst2d):
    N, D = table.shape
    KK, C = src2d.shape
    NW = 32
    NT = 16
    RPW = KK // NW
    SEG = 25
    NSEG = RPW // SEG
    RPT = N // NT
    ZF, ZR = RPT // C, RPT % C
    assert RPW * NW == KK and RPT * NT == N and D % L == 0
    assert NSEG * SEG == RPW and SEG == 25

    mesh = plsc.VectorSubcoreMesh(core_axis_name="c", subcore_axis_name="s")

    @functools.partial(
        pl.kernel, mesh=mesh,
        out_type=jax.ShapeDtypeStruct((2, N, D), jnp.float32),
        scratch_types=[
            pltpu.VMEM_SHARED((N, D), jnp.float32),
            pltpu.VMEM((SEG, C), jnp.int32),
            pltpu.VMEM((SEG, C), jnp.int32),
            pltpu.VMEM((SEG, C), jnp.float32),
            pltpu.VMEM((C, D), jnp.float32),
            pltpu.VMEM((C, D), jnp.float32),
            pltpu.VMEM((C, D), jnp.float32),
            pltpu.VMEM((C, D), jnp.float32),
            pltpu.SemaphoreType.DMA,
            pltpu.SemaphoreType.DMA,
            pltpu.SemaphoreType.DMA,
            pltpu.SemaphoreType.DMA,
            pltpu.SemaphoreType.DMA,
            pltpu.SemaphoreType.DMA,
            pltpu.SemaphoreType.DMA,
            pltpu.SemaphoreType.DMA,
        ],
        compiler_params=_sc_compiler_params(),
    )
    def layer_kernel(tab_h, w_h, src_h, dst_h, out_h,
                     acc, si_v, di_v, w_v, rb0, rb1, rb2, rb3,
                     g0, g1, g2, g3, s0, s1, s2, s3):
        cid = lax.axis_index("c")
        sid = lax.axis_index("s")
        wid = cid * NT + sid
        bufs = (rb0, rb1, rb2, rb3)
        gs = (g0, g1, g2, g3)
        ss = (s0, s1, s2, s3)

        zv = jnp.zeros((L,), jnp.float32)

        @pl.loop(0, C)
        def _(j):
            for t in range(D // L):
                rb0[j, pl.ds(t * L, L)] = zv

        for r in range(ZF):
            pltpu.async_copy(rb0, acc.at[pl.ds(sid * RPT + r * C, C)], g0)

        if ZR:
            pltpu.async_copy(rb0.at[pl.ds(0, ZR)],
                             acc.at[pl.ds(sid * RPT + ZF * C, ZR)], g0)

        for r in range(ZF):
            pltpu.make_async_copy(
                rb0, acc.at[pl.ds(sid * RPT + r * C, C)], g0).wait()

        if ZR:
            pltpu.make_async_copy(
                rb0.at[pl.ds(0, ZR)],
                acc.at[pl.ds(sid * RPT + ZF * C, ZR)], g0).wait()

        plsc.subcore_barrier()

        def scale(buf, r):
            @plsc.parallel_loop(0, C // L, unroll=2)
            def _(o):
                wg = w_v[r, pl.ds(o * L, L)]
                for e in range(L):
                    wv = jnp.full((L,), wg[e], jnp.float32)
                    row = o * L + e
                    for t in range(D // L):
                        sl = pl.ds(t * L, L)
                        buf[row, sl] = buf[row, sl] * wv

        def g_issue(q, bi):
            pltpu.async_copy(tab_h.at[si_v.at[q]], bufs[bi], gs[bi])

        def g_wait(q, bi):
            pltpu.make_async_copy(tab_h.at[si_v.at[q]], bufs[bi],
                                  gs[bi]).wait()

        def s_issue(q, bi):
            pltpu.async_copy(bufs[bi], acc.at[di_v.at[q]], ss[bi], add=True)

        def s_wait(q, bi):
            pltpu.make_async_copy(bufs[bi], acc.at[di_v.at[q]], ss[bi]).wait()

        def proc(q, bi):
            g_wait(q, bi)
            scale(bufs[bi], q)
            s_issue(q, bi)

        @pl.loop(0, NSEG)
        def _(s):
            seg0 = wid * RPW + s * SEG
            pltpu.async_copy(src_h.at[pl.ds(seg0, SEG)], si_v, g0)
            pltpu.async_copy(dst_h.at[pl.ds(seg0, SEG)], di_v, g1)
            pltpu.async_copy(w_h.at[pl.ds(seg0, SEG)], w_v, g2)
            pltpu.make_async_copy(src_h.at[pl.ds(seg0, SEG)], si_v, g0).wait()
            pltpu.make_async_copy(dst_h.at[pl.ds(seg0, SEG)], di_v, g1).wait()
            pltpu.make_async_copy(w_h.at[pl.ds(seg0, SEG)], w_v, g2).wait()

            g_issue(0, 0)
            g_issue(1, 1)

            @pl.loop(0, 5)
            def _(g):
                q0 = 4 * g
                proc(q0, 0)

                @pl.when(g > 0)
                def _():
                    s_wait(q0 - 2, 2)

                g_issue(q0 + 2, 2)
                proc(q0 + 1, 1)

                @pl.when(g > 0)
                def _():
                    s_wait(q0 - 1, 3)

                g_issue(q0 + 3, 3)
                proc(q0 + 2, 2)
                s_wait(q0, 0)
                g_issue(q0 + 4, 0)
                proc(q0 + 3, 3)
                s_wait(q0 + 1, 1)
                g_issue(q0 + 5, 1)

            proc(20, 0)
            s_wait(18, 2)
            g_issue(22, 2)
            proc(21, 1)
            s_wait(19, 3)
            g_issue(23, 3)
            proc(22, 2)
            s_wait(20, 0)
            g_issue(24, 0)
            proc(23, 3)
            proc(24, 0)
            s_wait(21, 1)
            s_wait(22, 2)
            s_wait(23, 3)
            s_wait(24, 0)

        plsc.subcore_barrier()
        pltpu.sync_copy(acc.at[pl.ds(sid * RPT, RPT)],
                        out_h.at[cid, pl.ds(sid * RPT, RPT)])

    return layer_kernel(table, w2d, src2d, dst2d)


def _tc_pre(h, d, w1t, b1r, wgd, wgs, bg):
    N, D = h.shape
    R = 1000
    G = N // R
    assert G * R == N

    def body(h_ref, d_ref, w1t_ref, b1_ref, wgd_ref, wgs_ref, bg_ref,
             hr_ref, hs_ref, a_ref, b_ref):
        x = jnp.dot(h_ref[...], w1t_ref[...], preferred_element_type=jnp.float32)
        hr = jnp.maximum(x + b1_ref[...], 0.0)
        hr_ref[...] = hr
        hs_ref[...] = hr * d_ref[...]
        a_ref[...] = jnp.dot(hr, wgd_ref[...],
                             preferred_element_type=jnp.float32) + bg_ref[0]
        b_ref[...] = jnp.dot(hr, wgs_ref[...],
                             preferred_element_type=jnp.float32)

    return pl.pallas_call(
        body,
        grid=(G,),
        in_specs=[
            pl.BlockSpec((R, D), lambda i: (i, 0)),
            pl.BlockSpec((R, 1), lambda i: (i, 0)),
            pl.BlockSpec((D, D), lambda i: (0, 0)),
            pl.BlockSpec((1, D), lambda i: (0, 0)),
            pl.BlockSpec((D, 1), lambda i: (0, 0)),
            pl.BlockSpec((D, 1), lambda i: (0, 0)),
            pl.BlockSpec(memory_space=pltpu.SMEM),
        ],
        out_specs=[
            pl.BlockSpec((R, D), lambda i: (i, 0)),
            pl.BlockSpec((R, D), lambda i: (i, 0)),
            pl.BlockSpec((R, 1), lambda i: (i, 0)),
            pl.BlockSpec((R, 1), lambda i: (i, 0)),
        ],
        out_shape=[
            jax.ShapeDtypeStruct((N, D), jnp.float32),
            jax.ShapeDtypeStruct((N, D), jnp.float32),
            jax.ShapeDtypeStruct((N, 1), jnp.float32),
            jax.ShapeDtypeStruct((N, 1), jnp.float32),
        ],
    )(h, d, w1t, b1r, wgd, wgs, bg)


def _tc_mid(acc, hr, d, wgd, wgs, bg):
    N, D = hr.shape
    R = 1000
    G = N // R

    def body(acc_ref, hr_ref, d_ref, wgd_ref, wgs_ref, bg_ref,
             hs_ref, a_ref, b_ref):
        dv = d_ref[...]
        h1 = EPS * hr_ref[...] + dv * (acc_ref[0] + acc_ref[1])
        hs_ref[...] = h1 * dv
        a_ref[...] = jnp.dot(h1, wgd_ref[...],
                             preferred_element_type=jnp.float32) + bg_ref[0]
        b_ref[...] = jnp.dot(h1, wgs_ref[...],
                             preferred_element_type=jnp.float32)

    return pl.pallas_call(
        body,
        grid=(G,),
        in_specs=[
            pl.BlockSpec((2, R, D), lambda i: (0, i, 0)),
            pl.BlockSpec((R, D), lambda i: (i, 0)),
            pl.BlockSpec((R, 1), lambda i: (i, 0)),
            pl.BlockSpec((D, 1), lambda i: (0, 0)),
            pl.BlockSpec((D, 1), lambda i: (0, 0)),
            pl.BlockSpec(memory_space=pltpu.SMEM),
        ],
        out_specs=[
            pl.BlockSpec((R, D), lambda i: (i, 0)),
            pl.BlockSpec((R, 1), lambda i: (i, 0)),
            pl.BlockSpec((R, 1), lambda i: (i, 0)),
        ],
        out_shape=[
            jax.ShapeDtypeStruct((N, D), jnp.float32),
            jax.ShapeDtypeStruct((N, 1), jnp.float32),
            jax.ShapeDtypeStruct((N, 1), jnp.float32),
        ],
    )(acc, hr, d, wgd, wgs, bg)


def _tc_post(acc, hr, d, w2t, b2r):
    N, D = hr.shape
    O = w2t.shape[1]
    R = 1000
    G = N // R

    def body(acc_ref, hr_ref, d_ref, w2t_ref, b2_ref, out_ref):
        h2 = EPS * hr_ref[...] + d_ref[...] * (acc_ref[0] + acc_ref[1])
        logits = jnp.dot(h2, w2t_ref[...],
                         preferred_element_type=jnp.float32) + b2_ref[...]
        m = jnp.max(logits, axis=1, keepdims=True)
        ex = jnp.exp(logits - m)
        s = jnp.sum(ex, axis=1, keepdims=True)
        out_ref[...] = logits - m - jnp.log(s)

    return pl.pallas_call(
        body,
        grid=(G,),
        in_specs=[
            pl.BlockSpec((2, R, D), lambda i: (0, i, 0)),
            pl.BlockSpec((R, D), lambda i: (i, 0)),
            pl.BlockSpec((R, 1), lambda i: (i, 0)),
            pl.BlockSpec((D, O), lambda i: (0, 0)),
            pl.BlockSpec((1, O), lambda i: (0, 0)),
        ],
        out_specs=pl.BlockSpec((R, O), lambda i: (i, 0)),
        out_shape=jax.ShapeDtypeStruct((N, O), jnp.float32),
    )(acc, hr, d, w2t, b2r)


def kernel(h, edge_index, W1, b1, Wg0, bg0, Wg1, bg1, W2, b2):
    N, D = h.shape
    E = edge_index.shape[1]
    src2d = edge_index[0].reshape(E // 80, 80)
    dst2d = edge_index[1].reshape(E // 80, 80)

    d16 = _sc_degree(dst2d, N)
    dcol = d16[:, :1]

    hr, hs0, a0, b0 = _tc_pre(
        h, dcol, W1.T, b1.reshape(1, -1),
        Wg0[0, :D].reshape(D, 1), Wg0[0, D:].reshape(D, 1), bg0)

    w1e = _sc_weights(a0[:, 0], b0[:, 0], src2d, dst2d)
    acc1 = _sc_layer(hs0, w1e, src2d, dst2d)

    hs1, a1, b1v = _tc_mid(
        acc1, hr, dcol, Wg1[0, :D].reshape(D, 1), Wg1[0, D:].reshape(D, 1), bg1)

    w2e = _sc_weights(a1[:, 0], b1v[:, 0], src2d, dst2d)
    acc2 = _sc_layer(hs1, w2e, src2d, dst2d)

    return _tc_post(acc2, hr, dcol, W2.T, b2.reshape(1, -1))


# speedup vs baseline: 1.0839x; 1.0839x over previous
import dataclasses
import functools

import jax
import jax.numpy as jnp
from jax import lax
from jax.experimental import pallas as pl
from jax.experimental.pallas import tpu as pltpu
from jax.experimental.pallas import tpu_sc as plsc

EPS = 0.3
L = 16


def _sc_compiler_params():
    cp = pltpu.CompilerParams()
    fields = pltpu.CompilerParams.__dataclass_fields__
    if "needs_layout_passes" in fields:
        cp = dataclasses.replace(cp, needs_layout_passes=False)
    if "use_tc_tiling_on_sc" in fields:
        cp = dataclasses.replace(cp, use_tc_tiling_on_sc=False)
    return cp


def _rsqrt_newton(x):
    i = plsc.bitcast(x, jnp.int32)
    i = jnp.int32(0x5F3759DF) - (i >> 1)
    y = plsc.bitcast(i, jnp.float32)
    for _ in range(3):
        y = y * (1.5 - 0.5 * x * y * y)
    return y


def _sc_degree(dst2d, n_nodes):
    KK, C = dst2d.shape
    E = KK * C
    NT = 16
    RPT2 = KK // NT
    FB = 25
    NPAD = ((n_nodes + NT * C - 1) // (NT * C)) * (NT * C)
    RPT = NPAD // NT
    OPT = n_nodes // NT
    assert RPT2 * NT == KK and OPT * NT == n_nodes and RPT2 % FB == 0

    mesh = plsc.VectorSubcoreMesh(core_axis_name="c", subcore_axis_name="s")

    @functools.partial(
        pl.kernel, mesh=mesh,
        out_type=jax.ShapeDtypeStruct((n_nodes, L), jnp.float32),
        scratch_types=[
            pltpu.VMEM_SHARED((NPAD, L), jnp.float32),
            pltpu.VMEM((RPT2, C), jnp.int32),
            pltpu.VMEM((C, L), jnp.float32),
            pltpu.VMEM((OPT, L), jnp.float32),
            pltpu.SemaphoreType.DMA,
        ],
        compiler_params=_sc_compiler_params(),
    )
    def deg_kernel(dst_h, out_h, acc, idx_v, ones_v, buf_v, sem):
        cid = lax.axis_index("c")
        sid = lax.axis_index("s")

        @pl.when(cid == 0)
        def _():
            zv = jnp.zeros((L,), jnp.float32)

            @pl.loop(0, C)
            def _(j):
                ones_v[j, :] = zv

            @pl.loop(0, RPT // C)
            def _(r):
                pltpu.sync_copy(ones_v, acc.at[pl.ds(sid * RPT + r * C, C)])

            ov = jnp.ones((L,), jnp.float32)

            @pl.loop(0, C)
            def _(j):
                ones_v[j, :] = ov

            pltpu.sync_copy(dst_h.at[pl.ds(sid * RPT2, RPT2)], idx_v)
            plsc.subcore_barrier()

            @pl.loop(0, RPT2 // FB)
            def _(g):
                for j in range(FB):
                    pltpu.async_copy(ones_v, acc.at[idx_v.at[g * FB + j]],
                                     sem, add=True)
                for j in range(FB):
                    pltpu.make_async_copy(
                        ones_v, acc.at[idx_v.at[g * FB + j]], sem).wait()

            plsc.subcore_barrier()
            pltpu.sync_copy(acc.at[pl.ds(sid * OPT, OPT)], buf_v)

            @pl.loop(0, OPT)
            def _(j):
                x = buf_v[j, :]
                x = jnp.maximum(x, 1.0)
                buf_v[j, :] = _rsqrt_newton(x)

            pltpu.sync_copy(buf_v, out_h.at[pl.ds(sid * OPT, OPT)])

    return deg_kernel(dst2d)


def _sc_weights(a, b, src2d, dst2d):
    N = a.shape[0]
    KK, C = src2d.shape
    NW = 32
    NT = 16
    RPW = KK // NW
    assert RPW * NW == KK

    mesh = plsc.VectorSubcoreMesh(core_axis_name="c", subcore_axis_name="s")

    @functools.partial(
        pl.kernel, mesh=mesh,
        out_type=jax.ShapeDtypeStruct((KK, C), jnp.float32),
        scratch_types=[
            pltpu.VMEM((N,), jnp.float32),
            pltpu.VMEM((N,), jnp.float32),
            pltpu.VMEM((RPW, C), jnp.int32),
            pltpu.VMEM((RPW, C), jnp.int32),
            pltpu.VMEM((RPW, C), jnp.float32),
            pltpu.SemaphoreType.DMA,
        ],
        compiler_params=_sc_compiler_params(),
    )
    def weights_kernel(a_h, b_h, src_h, dst_h, w_h,
                       a_v, b_v, si_v, di_v, w_v, sem):
        cid = lax.axis_index("c")
        sid = lax.axis_index("s")
        wid = cid * NT + sid
        cps = [
            pltpu.make_async_copy(a_h, a_v, sem),
            pltpu.make_async_copy(b_h, b_v, sem),
            pltpu.make_async_copy(src_h.at[pl.ds(wid * RPW, RPW)], si_v, sem),
            pltpu.make_async_copy(dst_h.at[pl.ds(wid * RPW, RPW)], di_v, sem),
        ]
        for cp in cps:
            cp.start()
        for cp in cps:
            cp.wait()

        @pl.loop(0, RPW)
        def _(r):
            for o in range(C // L):
                s16 = si_v[r, pl.ds(o * L, L)]
                t16 = di_v[r, pl.ds(o * L, L)]
                av = plsc.load_gather(a_v, [t16])
                bv = plsc.load_gather(b_v, [s16])
                w_v[r, pl.ds(o * L, L)] = (
                    1.0 - 2.0 / (1.0 + jnp.exp(2.0 * (av + bv))))

        pltpu.sync_copy(w_v, w_h.at[pl.ds(wid * RPW, RPW)])

    return weights_kernel(a, b, src2d, dst2d)


def _sc_layer(table, w2d, src2d, dst2d):
    N, D = table.shape
    KK, C = src2d.shape
    NW = 32
    NT = 16
    RPW = KK // NW
    SEG = 25
    NSEG = RPW // SEG
    RPT = N // NT
    ZF, ZR = RPT // C, RPT % C
    assert RPW * NW == KK and RPT * NT == N and D % L == 0
    assert NSEG * SEG == RPW and SEG == 25

    mesh = plsc.VectorSubcoreMesh(core_axis_name="c", subcore_axis_name="s")

    @functools.partial(
        pl.kernel, mesh=mesh,
        out_type=jax.ShapeDtypeStruct((2, N, D), jnp.float32),
        scratch_types=[
            pltpu.VMEM_SHARED((N, D), jnp.float32),
            pltpu.VMEM((SEG, C), jnp.int32),
            pltpu.VMEM((SEG, C), jnp.int32),
            pltpu.VMEM((SEG, C), jnp.float32),
            pltpu.VMEM((C, D), jnp.float32),
            pltpu.VMEM((C, D), jnp.float32),
            pltpu.VMEM((C, D), jnp.float32),
            pltpu.VMEM((C, D), jnp.float32),
            pltpu.SemaphoreType.DMA,
            pltpu.SemaphoreType.DMA,
            pltpu.SemaphoreType.DMA,
            pltpu.SemaphoreType.DMA,
            pltpu.SemaphoreType.DMA,
            pltpu.SemaphoreType.DMA,
            pltpu.SemaphoreType.DMA,
            pltpu.SemaphoreType.DMA,
        ],
        compiler_params=_sc_compiler_params(),
    )
    def layer_kernel(tab_h, w_h, src_h, dst_h, out_h,
                     acc, si_v, di_v, w_v, rb0, rb1, rb2, rb3,
                     g0, g1, g2, g3, s0, s1, s2, s3):
        cid = lax.axis_index("c")
        sid = lax.axis_index("s")
        wid = cid * NT + sid
        bufs = (rb0, rb1, rb2, rb3)
        gs = (g0, g1, g2, g3)
        ss = (s0, s1, s2, s3)

        zv = jnp.zeros((L,), jnp.float32)

        @pl.loop(0, C)
        def _(j):
            for t in range(D // L):
                rb0[j, pl.ds(t * L, L)] = zv

        for r in range(ZF):
            pltpu.async_copy(rb0, acc.at[pl.ds(sid * RPT + r * C, C)], g0)

        if ZR:
            pltpu.async_copy(rb0.at[pl.ds(0, ZR)],
                             acc.at[pl.ds(sid * RPT + ZF * C, ZR)], g0)

        for r in range(ZF):
            pltpu.make_async_copy(
                rb0, acc.at[pl.ds(sid * RPT + r * C, C)], g0).wait()

        if ZR:
            pltpu.make_async_copy(
                rb0.at[pl.ds(0, ZR)],
                acc.at[pl.ds(sid * RPT + ZF * C, ZR)], g0).wait()

        plsc.subcore_barrier()

        def scale(buf, r):
            @pl.loop(0, C // L)
            def _(o):
                wg = w_v[r, pl.ds(o * L, L)]
                for e in range(L):
                    wv = jnp.full((L,), wg[e], jnp.float32)
                    row = o * L + e
                    for t in range(D // L):
                        sl = pl.ds(t * L, L)
                        buf[row, sl] = buf[row, sl] * wv

        def g_issue(q, bi):
            pltpu.async_copy(tab_h.at[si_v.at[q]], bufs[bi], gs[bi])

        def g_wait(q, bi):
            pltpu.make_async_copy(tab_h.at[si_v.at[q]], bufs[bi],
                                  gs[bi]).wait()

        def s_issue(q, bi):
            pltpu.async_copy(bufs[bi], acc.at[di_v.at[q]], ss[bi], add=True)

        def s_wait(q, bi):
            pltpu.make_async_copy(bufs[bi], acc.at[di_v.at[q]], ss[bi]).wait()

        def proc(q, bi):
            g_wait(q, bi)
            scale(bufs[bi], q)
            s_issue(q, bi)

        @pl.loop(0, NSEG)
        def _(s):
            seg0 = wid * RPW + s * SEG
            pltpu.async_copy(src_h.at[pl.ds(seg0, SEG)], si_v, g0)
            pltpu.async_copy(dst_h.at[pl.ds(seg0, SEG)], di_v, g1)
            pltpu.async_copy(w_h.at[pl.ds(seg0, SEG)], w_v, g2)
            pltpu.make_async_copy(src_h.at[pl.ds(seg0, SEG)], si_v, g0).wait()
            pltpu.make_async_copy(dst_h.at[pl.ds(seg0, SEG)], di_v, g1).wait()
            pltpu.make_async_copy(w_h.at[pl.ds(seg0, SEG)], w_v, g2).wait()

            g_issue(0, 0)
            g_issue(1, 1)

            @pl.loop(0, 5)
            def _(g):
                q0 = 4 * g
                proc(q0, 0)

                @pl.when(g > 0)
                def _():
                    s_wait(q0 - 2, 2)

                g_issue(q0 + 2, 2)
                proc(q0 + 1, 1)

                @pl.when(g > 0)
                def _():
                    s_wait(q0 - 1, 3)

                g_issue(q0 + 3, 3)
                proc(q0 + 2, 2)
                s_wait(q0, 0)
                g_issue(q0 + 4, 0)
                proc(q0 + 3, 3)
                s_wait(q0 + 1, 1)
                g_issue(q0 + 5, 1)

            proc(20, 0)
            s_wait(18, 2)
            g_issue(22, 2)
            proc(21, 1)
            s_wait(19, 3)
            g_issue(23, 3)
            proc(22, 2)
            s_wait(20, 0)
            g_issue(24, 0)
            proc(23, 3)
            proc(24, 0)
            s_wait(21, 1)
            s_wait(22, 2)
            s_wait(23, 3)
            s_wait(24, 0)

        plsc.subcore_barrier()
        pltpu.sync_copy(acc.at[pl.ds(sid * RPT, RPT)],
                        out_h.at[cid, pl.ds(sid * RPT, RPT)])

    return layer_kernel(table, w2d, src2d, dst2d)


def _tc_pre(h, d, w1t, b1r, wgd, wgs, bg):
    N, D = h.shape
    R = 1000
    G = N // R
    assert G * R == N

    def body(h_ref, d_ref, w1t_ref, b1_ref, wgd_ref, wgs_ref, bg_ref,
             hr_ref, hs_ref, a_ref, b_ref):
        x = jnp.dot(h_ref[...], w1t_ref[...], preferred_element_type=jnp.float32)
        hr = jnp.maximum(x + b1_ref[...], 0.0)
        hr_ref[...] = hr
        hs_ref[...] = hr * d_ref[...]
        a_ref[...] = jnp.dot(hr, wgd_ref[...],
                             preferred_element_type=jnp.float32) + bg_ref[0]
        b_ref[...] = jnp.dot(hr, wgs_ref[...],
                             preferred_element_type=jnp.float32)

    return pl.pallas_call(
        body,
        grid=(G,),
        in_specs=[
            pl.BlockSpec((R, D), lambda i: (i, 0)),
            pl.BlockSpec((R, 1), lambda i: (i, 0)),
            pl.BlockSpec((D, D), lambda i: (0, 0)),
            pl.BlockSpec((1, D), lambda i: (0, 0)),
            pl.BlockSpec((D, 1), lambda i: (0, 0)),
            pl.BlockSpec((D, 1), lambda i: (0, 0)),
            pl.BlockSpec(memory_space=pltpu.SMEM),
        ],
        out_specs=[
            pl.BlockSpec((R, D), lambda i: (i, 0)),
            pl.BlockSpec((R, D), lambda i: (i, 0)),
            pl.BlockSpec((R, 1), lambda i: (i, 0)),
            pl.BlockSpec((R, 1), lambda i: (i, 0)),
        ],
        out_shape=[
            jax.ShapeDtypeStruct((N, D), jnp.float32),
            jax.ShapeDtypeStruct((N, D), jnp.float32),
            jax.ShapeDtypeStruct((N, 1), jnp.float32),
            jax.ShapeDtypeStruct((N, 1), jnp.float32),
        ],
    )(h, d, w1t, b1r, wgd, wgs, bg)


def _tc_mid(acc, hr, d, wgd, wgs, bg):
    N, D = hr.shape
    R = 1000
    G = N // R

    def body(acc_ref, hr_ref, d_ref, wgd_ref, wgs_ref, bg_ref,
             hs_ref, a_ref, b_ref):
        dv = d_ref[...]
        h1 = EPS * hr_ref[...] + dv * (acc_ref[0] + acc_ref[1])
        hs_ref[...] = h1 * dv
        a_ref[...] = jnp.dot(h1, wgd_ref[...],
                             preferred_element_type=jnp.float32) + bg_ref[0]
        b_ref[...] = jnp.dot(h1, wgs_ref[...],
                             preferred_element_type=jnp.float32)

    return pl.pallas_call(
        body,
        grid=(G,),
        in_specs=[
            pl.BlockSpec((2, R, D), lambda i: (0, i, 0)),
            pl.BlockSpec((R, D), lambda i: (i, 0)),
            pl.BlockSpec((R, 1), lambda i: (i, 0)),
            pl.BlockSpec((D, 1), lambda i: (0, 0)),
            pl.BlockSpec((D, 1), lambda i: (0, 0)),
            pl.BlockSpec(memory_space=pltpu.SMEM),
        ],
        out_specs=[
            pl.BlockSpec((R, D), lambda i: (i, 0)),
            pl.BlockSpec((R, 1), lambda i: (i, 0)),
            pl.BlockSpec((R, 1), lambda i: (i, 0)),
        ],
        out_shape=[
            jax.ShapeDtypeStruct((N, D), jnp.float32),
            jax.ShapeDtypeStruct((N, 1), jnp.float32),
            jax.ShapeDtypeStruct((N, 1), jnp.float32),
        ],
    )(acc, hr, d, wgd, wgs, bg)


def _tc_post(acc, hr, d, w2t, b2r):
    N, D = hr.shape
    O = w2t.shape[1]
    R = 1000
    G = N // R

    def body(acc_ref, hr_ref, d_ref, w2t_ref, b2_ref, out_ref):
        h2 = EPS * hr_ref[...] + d_ref[...] * (acc_ref[0] + acc_ref[1])
        logits = jnp.dot(h2, w2t_ref[...],
                         preferred_element_type=jnp.float32) + b2_ref[...]
        m = jnp.max(logits, axis=1, keepdims=True)
        ex = jnp.exp(logits - m)
        s = jnp.sum(ex, axis=1, keepdims=True)
        out_ref[...] = logits - m - jnp.log(s)

    return pl.pallas_call(
        body,
        grid=(G,),
        in_specs=[
            pl.BlockSpec((2, R, D), lambda i: (0, i, 0)),
            pl.BlockSpec((R, D), lambda i: (i, 0)),
            pl.BlockSpec((R, 1), lambda i: (i, 0)),
            pl.BlockSpec((D, O), lambda i: (0, 0)),
            pl.BlockSpec((1, O), lambda i: (0, 0)),
        ],
        out_specs=pl.BlockSpec((R, O), lambda i: (i, 0)),
        out_shape=jax.ShapeDtypeStruct((N, O), jnp.float32),
    )(acc, hr, d, w2t, b2r)


def kernel(h, edge_index, W1, b1, Wg0, bg0, Wg1, bg1, W2, b2):
    N, D = h.shape
    E = edge_index.shape[1]
    src2d = edge_index[0].reshape(E // 80, 80)
    dst2d = edge_index[1].reshape(E // 80, 80)

    d16 = _sc_degree(dst2d, N)
    dcol = d16[:, :1]

    hr, hs0, a0, b0 = _tc_pre(
        h, dcol, W1.T, b1.reshape(1, -1),
        Wg0[0, :D].reshape(D, 1), Wg0[0, D:].reshape(D, 1), bg0)

    w1e = _sc_weights(a0[:, 0], b0[:, 0], src2d, dst2d)
    acc1 = _sc_layer(hs0, w1e, src2d, dst2d)

    hs1, a1, b1v = _tc_mid(
        acc1, hr, dcol, Wg1[0, :D].reshape(D, 1), Wg1[0, D:].reshape(D, 1), bg1)

    w2e = _sc_weights(a1[:, 0], b1v[:, 0], src2d, dst2d)
    acc2 = _sc_layer(hs1, w2e, src2d, dst2d)

    return _tc_post(acc2, hr, dcol, W2.T, b2.reshape(1, -1))
